# baseline (speedup 1.0000x reference)
import functools

import jax
import jax.numpy as jnp
from jax import lax
from jax.experimental import pallas as pl
from jax.experimental.pallas import tpu as pltpu
from jax.experimental.pallas import tpu_sc as plsc

N_NODES = 10000
C_IN = 128
NC, NS = 2, 16
NW = NC * NS
S_TOTAL = 160000
EDGES_PER_W = S_TOTAL // NW
CHUNK = 128
N_CH = EDGES_PER_W // CHUNK
REM = EDGES_PER_W - N_CH * CHUNK
STRIPE = 624
DROWS = 80
L = 16


def _sc_body(x3_hbm, sels_hbm, ei_hbm, outp_hbm, outd_hbm,
             acc, dega, sel2, src3, dst3, rows2, hist, idx80,
             src_r, dst_r8, dst_r16, rows_r, lsem, isem, gsem, ssem):
    c = lax.axis_index("c")
    s = lax.axis_index("s")
    wid = c * NS + s
    base = wid * EDGES_PER_W
    ones = jnp.ones((L,), jnp.float32)
    src_hbm = ei_hbm.at[0]
    dst_hbm = ei_hbm.at[1]
    x_hbm = x3_hbm.at[0]

    def zero_rows(i, _):
        rows2[0, i // (C_IN // L), pl.ds((i % (C_IN // L)) * L, L)] = (
            jnp.zeros((L,), jnp.float32))
        return 0
    lax.fori_loop(0, CHUNK * (C_IN // L), zero_rows, 0)
    r0 = s * STRIPE
    for j in range(5):
        pltpu.sync_copy(rows2.at[0], acc.at[pl.ds(r0 + j * CHUNK, CHUNK)])
    def zero_hist(i, _):
        hist[i // (C_IN // L), pl.ds((i % (C_IN // L)) * L, L)] = (
            jnp.zeros((L,), jnp.float32))
        return 0
    lax.fori_loop(0, DROWS * (C_IN // L), zero_hist, 0)

    @pl.when(s == 0)
    def _():
        pltpu.sync_copy(rows2.at[0, pl.ds(0, DROWS)], dega)
    for k in range(DROWS // L):
        idx80[pl.ds(k * L, L)] = lax.iota(jnp.int32, L) + k * L
    dst_r16[...] = jnp.zeros((L,), jnp.int32)
    plsc.subcore_barrier()

    def drain(sem, src_d, dst_d):
        pltpu.make_async_copy(src_d, dst_d, sem).wait()

    pltpu.sync_copy(sels_hbm.at[pl.ds(base, CHUNK)], sel2.at[0])
    pltpu.sync_copy(sels_hbm.at[pl.ds(base + CHUNK, CHUNK)], sel2.at[1])
    pltpu.sync_copy(src_hbm.at[sel2.at[0]], src3.at[0])
    pltpu.sync_copy(dst_hbm.at[sel2.at[0]], dst3.at[0])
    pltpu.async_copy(src_hbm.at[sel2.at[1]], src3.at[1], isem)
    pltpu.async_copy(dst_hbm.at[sel2.at[1]], dst3.at[1], isem)
    pltpu.async_copy(sels_hbm.at[pl.ds(base + 2 * CHUNK, CHUNK)], sel2.at[0],
                     lsem)
    pltpu.async_copy(x_hbm.at[src3.at[0]], rows2.at[0], gsem)

    def body(i, _):
        b = lax.rem(i, 2)
        b1 = lax.rem(i + 1, 2)
        s0 = lax.rem(i, 3)
        s3 = lax.rem(i + 2, 3)
        drain(gsem, x_hbm.at[pl.ds(0, CHUNK)], rows2.at[b])
        pltpu.async_copy(rows2.at[b], acc.at[dst3.at[s0]], ssem, add=True)

        for k in range(CHUNK // L):
            dv = dst3[s0, pl.ds(k * L, L)]
            plsc.addupdate_scatter(
                hist, [lax.shift_right_logical(dv, 7),
                       lax.bitwise_and(dv, 127)], ones)

        @pl.when(i <= N_CH - 2)
        def _():
            drain(isem, src_hbm.at[pl.ds(0, CHUNK)], src3.at[0])
            drain(isem, src_hbm.at[pl.ds(0, CHUNK)], src3.at[0])

        @pl.when(i >= 1)
        def _():
            drain(ssem, x_hbm.at[pl.ds(0, CHUNK)], rows2.at[0])

        @pl.when(i <= N_CH - 2)
        def _():
            pltpu.async_copy(x_hbm.at[src3.at[lax.rem(i + 1, 3)]],
                             rows2.at[b1], gsem)

        @pl.when(i <= N_CH - 3)
        def _():
            drain(lsem, sels_hbm.at[pl.ds(0, CHUNK)], sel2.at[0])
            pltpu.async_copy(src_hbm.at[sel2.at[b]], src3.at[s3], isem)
            pltpu.async_copy(dst_hbm.at[sel2.at[b]], dst3.at[s3], isem)

        @pl.when(i <= N_CH - 4)
        def _():
            pltpu.async_copy(
                sels_hbm.at[pl.ds(base + (i + 3) * CHUNK, CHUNK)],
                sel2.at[b1], lsem)
        return 0
    lax.fori_loop(0, N_CH, body, 0)

    drain(ssem, x_hbm.at[pl.ds(0, CHUNK)], rows2.at[0])

    o = base + N_CH * CHUNK
    pltpu.sync_copy(sels_hbm.at[pl.ds(o, REM)], sel2.at[0, pl.ds(0, REM)])
    pltpu.sync_copy(src_hbm.at[sel2.at[0, pl.ds(0, REM)]], src_r)
    pltpu.sync_copy(dst_hbm.at[sel2.at[0, pl.ds(0, REM)]], dst_r8)
    pltpu.sync_copy(dst_hbm.at[sel2.at[0, pl.ds(0, REM)]],
                    dst_r16.at[pl.ds(0, REM)])
    pltpu.sync_copy(x_hbm.at[src_r], rows_r)
    pltpu.sync_copy(rows_r, acc.at[dst_r8], add=True)
    dv = dst_r16[...]
    plsc.addupdate_scatter(
        hist, [lax.shift_right_logical(dv, 7), lax.bitwise_and(dv, 127)],
        ones, mask=lax.iota(jnp.int32, L) < REM)

    pltpu.sync_copy(hist, dega.at[idx80], add=True)

    plsc.subcore_barrier()
    for j in range(4):
        pltpu.sync_copy(acc.at[pl.ds(r0 + j * CHUNK, CHUNK)],
                        outp_hbm.at[c, pl.ds(r0 + j * CHUNK, CHUNK)])
    pltpu.sync_copy(acc.at[pl.ds(r0 + 512, 112)],
                    outp_hbm.at[c, pl.ds(r0 + 512, 112)])

    @pl.when(s == NS - 1)
    def _():
        pltpu.sync_copy(acc.at[pl.ds(9984, 16)],
                        outp_hbm.at[c, pl.ds(9984, 16)])

    @pl.when(s == 0)
    def _():
        pltpu.sync_copy(dega, outd_hbm.at[c])


_sc_agg = functools.partial(
    pl.kernel,
    out_type=(jax.ShapeDtypeStruct((NC, N_NODES, C_IN), jnp.float32),
              jax.ShapeDtypeStruct((NC, DROWS, C_IN), jnp.float32)),
    mesh=plsc.VectorSubcoreMesh(core_axis_name="c", subcore_axis_name="s",
                                num_cores=NC, num_subcores=NS),
    compiler_params=pltpu.CompilerParams(use_tc_tiling_on_sc=False,
                                         needs_layout_passes=False),
    scratch_types=(
        pltpu.VMEM_SHARED((N_NODES, C_IN), jnp.float32),
        pltpu.VMEM_SHARED((DROWS, C_IN), jnp.float32),
        pltpu.VMEM((2, CHUNK), jnp.int32),
        pltpu.VMEM((3, CHUNK), jnp.int32),
        pltpu.VMEM((3, CHUNK), jnp.int32),
        pltpu.VMEM((2, CHUNK, C_IN), jnp.float32),
        pltpu.VMEM((DROWS, C_IN), jnp.float32),
        pltpu.VMEM((DROWS,), jnp.int32),
        pltpu.VMEM((REM,), jnp.int32),
        pltpu.VMEM((REM,), jnp.int32),
        pltpu.VMEM((L,), jnp.int32),
        pltpu.VMEM((REM, C_IN), jnp.float32),
        pltpu.SemaphoreType.DMA,
        pltpu.SemaphoreType.DMA,
        pltpu.SemaphoreType.DMA,
        pltpu.SemaphoreType.DMA,
    ),
)(_sc_body)


RB = 1024


def _tc_body(p_ref, d_ref, wz_ref, bz_ref, wh_ref, bh_ref,
             lz_ref, lzb_ref, lh_ref, lhb_ref, out_ref):
    acc = p_ref[0] + p_ref[1]
    dblk = d_ref[0] + d_ref[1]
    rown = lax.broadcasted_iota(jnp.int32, (RB, 8), 0) >> 7
    e8 = (rown == lax.broadcasted_iota(jnp.int32, (RB, 8), 1)).astype(
        jnp.float32)
    rowsel = jnp.dot(e8, dblk, preferred_element_type=jnp.float32,
                     precision=lax.Precision.HIGHEST)
    lanes = lax.broadcasted_iota(jnp.int32, (RB, C_IN), 0) & (C_IN - 1)
    osel = (lanes == lax.broadcasted_iota(jnp.int32, (RB, C_IN), 1)).astype(
        jnp.float32)
    deg = jnp.sum(rowsel * osel, axis=1, keepdims=True)
    agg = acc * (1.0 / jnp.maximum(deg, 1.0))
    az = jnp.dot(agg, wz_ref[...], preferred_element_type=jnp.float32) + bz_ref[...]
    ah = jnp.dot(agg, wh_ref[...], preferred_element_type=jnp.float32) + bh_ref[...]
    z = jax.nn.sigmoid(
        jnp.dot(az, lz_ref[...], preferred_element_type=jnp.float32) + lzb_ref[...])
    ht = jnp.tanh(
        jnp.dot(ah, lh_ref[...], preferred_element_type=jnp.float32) + lhb_ref[...])
    out_ref[0] = (1.0 - z) * ht


def _tc_gates(partials, deg, Wz, bz, Wh, bh, Lz_top, Lzb, Lh_top, Lhb):
    grid = (N_NODES + RB - 1) // RB
    w_spec = pl.BlockSpec((C_IN, C_IN), lambda i: (0, 0))
    b_spec = pl.BlockSpec((1, C_IN), lambda i: (0, 0))
    return pl.pallas_call(
        _tc_body,
        grid=(grid,),
        in_specs=[
            pl.BlockSpec((NC, RB, C_IN), lambda i: (0, i, 0)),
            pl.BlockSpec((NC, RB // C_IN, C_IN), lambda i: (0, i, 0)),
            w_spec, b_spec, w_spec, b_spec,
            w_spec, b_spec, w_spec, b_spec,
        ],
        out_specs=pl.BlockSpec((1, RB, C_IN), lambda i: (0, i, 0)),
        out_shape=jax.ShapeDtypeStruct((1, N_NODES, C_IN), jnp.float32),
    )(partials, deg, Wz, bz, Wh, bh, Lz_top, Lzb, Lh_top, Lhb)


@jax.jit
def kernel(X, edge_index, selections, Wz, bz, Wr, br, Wh, bh,
           Lz, Lzb, Lr, Lrb, Lh, Lhb):
    sels = selections.astype(jnp.int32)
    ei = edge_index.astype(jnp.int32)

    partials, degp = _sc_agg(X, sels, ei)

    return _tc_gates(partials, degp, Wz, bz.reshape(1, C_IN),
                     Wh, bh.reshape(1, C_IN),
                     Lz[:C_IN], Lzb.reshape(1, C_IN),
                     Lh[:C_IN], Lhb.reshape(1, C_IN))


# speedup vs baseline: 1.0368x; 1.0368x over previous
import functools

import jax
import jax.numpy as jnp
from jax import lax
from jax.experimental import pallas as pl
from jax.experimental.pallas import tpu as pltpu
from jax.experimental.pallas import tpu_sc as plsc

N_NODES = 10000
C_IN = 128
NC, NS = 2, 16
NW = NC * NS
S_TOTAL = 160000
EDGES_PER_W = S_TOTAL // NW
CHUNK = 128
N_CH = EDGES_PER_W // CHUNK
REM = EDGES_PER_W - N_CH * CHUNK
STRIPE = 624
DROWS = 80
L = 16


def _sc_body(x3_hbm, sels_hbm, ei_hbm, outp_hbm, outd_hbm,
             acc, dega, sel2, src3, dst3, rows2, hist, idx80,
             src_r, dst_r8, dst_r16, rows_r, lsem, isem, gsem, ssem):
    c = lax.axis_index("c")
    s = lax.axis_index("s")
    wid = c * NS + s
    base = wid * EDGES_PER_W
    ones = jnp.ones((L,), jnp.float32)
    src_hbm = ei_hbm.at[0]
    dst_hbm = ei_hbm.at[1]
    x_hbm = x3_hbm.at[0]

    def zero_rows(i, _):
        rows2[0, i // (C_IN // L), pl.ds((i % (C_IN // L)) * L, L)] = (
            jnp.zeros((L,), jnp.float32))
        return 0
    lax.fori_loop(0, CHUNK * (C_IN // L), zero_rows, 0)
    r0 = s * STRIPE
    for j in range(5):
        pltpu.sync_copy(rows2.at[0], acc.at[pl.ds(r0 + j * CHUNK, CHUNK)])
    def zero_hist(i, _):
        hist[i // (C_IN // L), pl.ds((i % (C_IN // L)) * L, L)] = (
            jnp.zeros((L,), jnp.float32))
        return 0
    lax.fori_loop(0, DROWS * (C_IN // L), zero_hist, 0)

    @pl.when(s == 0)
    def _():
        pltpu.sync_copy(rows2.at[0, pl.ds(0, DROWS)], dega)
    for k in range(DROWS // L):
        idx80[pl.ds(k * L, L)] = lax.iota(jnp.int32, L) + k * L
    dst_r16[...] = jnp.zeros((L,), jnp.int32)
    plsc.subcore_barrier()

    def drain(sem, src_d, dst_d):
        pltpu.make_async_copy(src_d, dst_d, sem).wait()

    pltpu.sync_copy(sels_hbm.at[pl.ds(base, CHUNK)], sel2.at[0])
    pltpu.sync_copy(sels_hbm.at[pl.ds(base + CHUNK, CHUNK)], sel2.at[1])
    pltpu.sync_copy(src_hbm.at[sel2.at[0]], src3.at[0])
    pltpu.sync_copy(dst_hbm.at[sel2.at[0]], dst3.at[0])
    pltpu.async_copy(src_hbm.at[sel2.at[1]], src3.at[1], isem)
    pltpu.async_copy(dst_hbm.at[sel2.at[1]], dst3.at[1], isem)
    pltpu.async_copy(sels_hbm.at[pl.ds(base + 2 * CHUNK, CHUNK)], sel2.at[0],
                     lsem)
    pltpu.async_copy(x_hbm.at[src3.at[0]], rows2.at[0], gsem)

    def body(i, _):
        b = lax.rem(i, 2)
        b1 = lax.rem(i + 1, 2)
        s0 = lax.rem(i, 3)
        s3 = lax.rem(i + 2, 3)
        drain(gsem, x_hbm.at[pl.ds(0, CHUNK)], rows2.at[b])
        pltpu.async_copy(rows2.at[b], acc.at[dst3.at[s0]], ssem, add=True)

        for k in range(CHUNK // L):
            dv = dst3[s0, pl.ds(k * L, L)]
            plsc.addupdate_scatter(
                hist, [lax.shift_right_logical(dv, 7),
                       lax.bitwise_and(dv, 127)], ones)

        @pl.when(i <= N_CH - 2)
        def _():
            drain(isem, src_hbm.at[pl.ds(0, CHUNK)], src3.at[0])
            drain(isem, src_hbm.at[pl.ds(0, CHUNK)], src3.at[0])

        @pl.when(i >= 1)
        def _():
            drain(ssem, x_hbm.at[pl.ds(0, CHUNK)], rows2.at[0])

        @pl.when(i <= N_CH - 2)
        def _():
            pltpu.async_copy(x_hbm.at[src3.at[lax.rem(i + 1, 3)]],
                             rows2.at[b1], gsem)

        @pl.when(i <= N_CH - 3)
        def _():
            drain(lsem, sels_hbm.at[pl.ds(0, CHUNK)], sel2.at[0])
            pltpu.async_copy(src_hbm.at[sel2.at[b]], src3.at[s3], isem)
            pltpu.async_copy(dst_hbm.at[sel2.at[b]], dst3.at[s3], isem)

        @pl.when(i <= N_CH - 4)
        def _():
            pltpu.async_copy(
                sels_hbm.at[pl.ds(base + (i + 3) * CHUNK, CHUNK)],
                sel2.at[b1], lsem)
        return 0
    lax.fori_loop(0, N_CH, body, 0)

    drain(ssem, x_hbm.at[pl.ds(0, CHUNK)], rows2.at[0])

    o = base + N_CH * CHUNK
    pltpu.sync_copy(sels_hbm.at[pl.ds(o, REM)], sel2.at[0, pl.ds(0, REM)])
    pltpu.sync_copy(src_hbm.at[sel2.at[0, pl.ds(0, REM)]], src_r)
    pltpu.sync_copy(dst_hbm.at[sel2.at[0, pl.ds(0, REM)]], dst_r8)
    pltpu.sync_copy(dst_hbm.at[sel2.at[0, pl.ds(0, REM)]],
                    dst_r16.at[pl.ds(0, REM)])
    pltpu.sync_copy(x_hbm.at[src_r], rows_r)
    pltpu.sync_copy(rows_r, acc.at[dst_r8], add=True)
    dv = dst_r16[...]
    plsc.addupdate_scatter(
        hist, [lax.shift_right_logical(dv, 7), lax.bitwise_and(dv, 127)],
        ones, mask=lax.iota(jnp.int32, L) < REM)

    pltpu.sync_copy(hist, dega.at[idx80], add=True)

    plsc.subcore_barrier()
    for j in range(4):
        pltpu.sync_copy(acc.at[pl.ds(r0 + j * CHUNK, CHUNK)],
                        outp_hbm.at[c, pl.ds(r0 + j * CHUNK, CHUNK)])
    pltpu.sync_copy(acc.at[pl.ds(r0 + 512, 112)],
                    outp_hbm.at[c, pl.ds(r0 + 512, 112)])

    @pl.when(s == NS - 1)
    def _():
        pltpu.sync_copy(acc.at[pl.ds(9984, 16)],
                        outp_hbm.at[c, pl.ds(9984, 16)])

    @pl.when(s == 0)
    def _():
        pltpu.sync_copy(dega, outd_hbm.at[c])


_sc_agg = functools.partial(
    pl.kernel,
    out_type=(jax.ShapeDtypeStruct((NC, N_NODES, C_IN), jnp.float32),
              jax.ShapeDtypeStruct((NC, DROWS, C_IN), jnp.float32)),
    mesh=plsc.VectorSubcoreMesh(core_axis_name="c", subcore_axis_name="s",
                                num_cores=NC, num_subcores=NS),
    compiler_params=pltpu.CompilerParams(use_tc_tiling_on_sc=False,
                                         needs_layout_passes=False),
    scratch_types=(
        pltpu.VMEM_SHARED((N_NODES, C_IN), jnp.float32),
        pltpu.VMEM_SHARED((DROWS, C_IN), jnp.float32),
        pltpu.VMEM((2, CHUNK), jnp.int32),
        pltpu.VMEM((3, CHUNK), jnp.int32),
        pltpu.VMEM((3, CHUNK), jnp.int32),
        pltpu.VMEM((2, CHUNK, C_IN), jnp.float32),
        pltpu.VMEM((DROWS, C_IN), jnp.float32),
        pltpu.VMEM((DROWS,), jnp.int32),
        pltpu.VMEM((REM,), jnp.int32),
        pltpu.VMEM((REM,), jnp.int32),
        pltpu.VMEM((L,), jnp.int32),
        pltpu.VMEM((REM, C_IN), jnp.float32),
        pltpu.SemaphoreType.DMA,
        pltpu.SemaphoreType.DMA,
        pltpu.SemaphoreType.DMA,
        pltpu.SemaphoreType.DMA,
    ),
)(_sc_body)


RB = 1024


def _tc_body(p_ref, d_ref, wz_ref, bz_ref, wh_ref, bh_ref,
             lz_ref, lzb_ref, lh_ref, lhb_ref, out_ref):
    acc = p_ref[0] + p_ref[1]
    dblk = d_ref[0] + d_ref[1]
    rown = lax.broadcasted_iota(jnp.int32, (RB, 8), 0) >> 7
    e8 = (rown == lax.broadcasted_iota(jnp.int32, (RB, 8), 1)).astype(
        jnp.float32)
    rowsel = jnp.dot(e8, dblk, preferred_element_type=jnp.float32)
    lanes = lax.broadcasted_iota(jnp.int32, (RB, C_IN), 0) & (C_IN - 1)
    osel = (lanes == lax.broadcasted_iota(jnp.int32, (RB, C_IN), 1)).astype(
        jnp.float32)
    deg = jnp.sum(rowsel * osel, axis=1, keepdims=True)
    agg = acc * (1.0 / jnp.maximum(deg, 1.0))
    az = jnp.dot(agg, wz_ref[...], preferred_element_type=jnp.float32) + bz_ref[...]
    ah = jnp.dot(agg, wh_ref[...], preferred_element_type=jnp.float32) + bh_ref[...]
    z = jax.nn.sigmoid(
        jnp.dot(az, lz_ref[...], preferred_element_type=jnp.float32) + lzb_ref[...])
    ht = jnp.tanh(
        jnp.dot(ah, lh_ref[...], preferred_element_type=jnp.float32) + lhb_ref[...])
    out_ref[0] = (1.0 - z) * ht


def _tc_gates(partials, deg, Wz, bz, Wh, bh, Lz_top, Lzb, Lh_top, Lhb):
    grid = (N_NODES + RB - 1) // RB
    w_spec = pl.BlockSpec((C_IN, C_IN), lambda i: (0, 0))
    b_spec = pl.BlockSpec((1, C_IN), lambda i: (0, 0))
    return pl.pallas_call(
        _tc_body,
        grid=(grid,),
        in_specs=[
            pl.BlockSpec((NC, RB, C_IN), lambda i: (0, i, 0)),
            pl.BlockSpec((NC, RB // C_IN, C_IN), lambda i: (0, i, 0)),
            w_spec, b_spec, w_spec, b_spec,
            w_spec, b_spec, w_spec, b_spec,
        ],
        out_specs=pl.BlockSpec((1, RB, C_IN), lambda i: (0, i, 0)),
        out_shape=jax.ShapeDtypeStruct((1, N_NODES, C_IN), jnp.float32),
    )(partials, deg, Wz, bz, Wh, bh, Lz_top, Lzb, Lh_top, Lhb)


@jax.jit
def kernel(X, edge_index, selections, Wz, bz, Wr, br, Wh, bh,
           Lz, Lzb, Lr, Lrb, Lh, Lhb):
    sels = selections.astype(jnp.int32)
    ei = edge_index.astype(jnp.int32)

    partials, degp = _sc_agg(X, sels, ei)

    return _tc_gates(partials, degp, Wz, bz.reshape(1, C_IN),
                     Wh, bh.reshape(1, C_IN),
                     Lz[:C_IN], Lzb.reshape(1, C_IN),
                     Lh[:C_IN], Lhb.reshape(1, C_IN))
